# baseline (speedup 1.0000x reference)
import functools

import jax
import jax.numpy as jnp
from jax import lax
from jax.experimental import pallas as pl
from jax.experimental.pallas import tpu as pltpu
from jax.experimental.pallas import tpu_sc as plsc

N = 10000
E = 320000
D = 128
N_RBF = 20
N_GRAPHS = 64
NUM_LAYERS = 3

NC = 2
NS = 16
NW = NC * NS
EPT = E // NW

ROWS_PT = N // NS

NHALF = N // NC
SROWS = NHALF + 8
SROWS_PT = SROWS // NS
EPT_S = E // NS
CS = 80
NCH_S = EPT_S // CS
CP = 80
NCH_P = EPT // CP

BR = 1000
GRID = N // BR

_F32 = jnp.float32
_I32 = jnp.int32


def _bf16_round(v):
    i = plsc.bitcast(v, _I32)
    r = i + 0x8000 + (lax.shift_right_logical(i, jnp.full((16,), 16, _I32))
                      & jnp.full((16,), 1, _I32))
    return plsc.bitcast(r & jnp.full((16,), -65536, _I32), _F32)


def _sc_mesh():
    return plsc.VectorSubcoreMesh(core_axis_name="c", subcore_axis_name="s")


NPAD = 10240


def _sc_preproc_body(src_hbm, dst_hbm, px_hbm, py_hbm, pz_hbm, ea_hbm,
                     zeros_hbm, out_hbm,
                     sidx_a, sidx_b, didx_a, didx_b, didx2a, didx2b,
                     px, py, pz, ea_a, ea_b,
                     F_a, F_b, G, sem_ea, sem_eb, sem_sa, sem_sb):
    c = lax.axis_index("c")
    s = lax.axis_index("s")
    base_row = s * SROWS_PT
    bound = c * NHALF
    ebase = s * EPT_S

    pltpu.sync_copy(zeros_hbm, G.at[pl.ds(base_row, SROWS_PT)])
    pltpu.sync_copy(px_hbm, px)
    pltpu.sync_copy(py_hbm, py)
    pltpu.sync_copy(pz_hbm, pz)
    zeros16 = jnp.zeros((16,), _F32)

    def zero_body(r, carry):
        for blk in range(3, 8):
            F_a[r, pl.ds(16 * blk, 16)] = zeros16
            F_b[r, pl.ds(16 * blk, 16)] = zeros16
        return carry

    lax.fori_loop(0, CS, zero_body, None)
    plsc.subcore_barrier()

    step = 4.0 / (N_RBF - 1)
    iota16 = lax.iota(_I32, 16)
    shift1 = jnp.full((16,), 1, _I32)
    magic = jnp.full((16,), 0x5F3759DF, _I32)
    c_lo = iota16.astype(_F32) * step
    c_hi = (iota16.astype(_F32) + 16.0) * step
    hi_mask = iota16 < (N_RBF - 16)

    def stage(j, sbuf, dbuf, ebuf, sem):
        pltpu.make_async_copy(
            src_hbm.at[pl.ds(ebase + j * CS, CS)], sbuf, sem).start()
        pltpu.make_async_copy(
            dst_hbm.at[pl.ds(ebase + j * CS, CS)], dbuf, sem).start()
        pltpu.make_async_copy(
            ea_hbm.at[pl.ds(ebase + j * CS, CS)], ebuf, sem).start()

    def stage_wait(j, sbuf, dbuf, ebuf, sem):
        pltpu.make_async_copy(
            src_hbm.at[pl.ds(ebase + j * CS, CS)], sbuf, sem).wait()
        pltpu.make_async_copy(
            dst_hbm.at[pl.ds(ebase + j * CS, CS)], dbuf, sem).wait()
        pltpu.make_async_copy(
            ea_hbm.at[pl.ds(ebase + j * CS, CS)], ebuf, sem).wait()

    def compute(j, sidx_buf, didx_buf, ea_buf, F_buf, didx2_buf):
        for g in range(CS // 16):
            sidxv = sidx_buf[pl.ds(g * 16, 16)]
            didxv = didx_buf[pl.ds(g * 16, 16)]
            dx = plsc.load_gather(px, [sidxv]) - plsc.load_gather(px, [didxv])
            dy = plsc.load_gather(py, [sidxv]) - plsc.load_gather(py, [didxv])
            dz = plsc.load_gather(pz, [sidxv]) - plsc.load_gather(pz, [didxv])
            d2v = dx * dx + dy * dy + dz * dz + 1e-12
            bits = plsc.bitcast(d2v, _I32)
            bits = magic - lax.shift_right_logical(bits, shift1)
            y = plsc.bitcast(bits, _F32)
            for _ in range(2):
                y = y * (1.5 - 0.5 * d2v * y * y)
            distv = d2v * y
            d = didxv - bound
            oob = (d < 0) | (d >= NHALF)
            didx2_buf[0, pl.ds(g * 16, 16)] = jnp.where(oob, NHALF, d)
            for e in range(16):
                r = g * 16 + e
                F_buf[r, pl.ds(0, 16)] = _bf16_round(ea_buf[r, pl.ds(0, 16)])
                dv = jnp.broadcast_to(distv[e], (16,))
                t_lo = dv - c_lo
                F_buf[r, pl.ds(16, 16)] = _bf16_round(jnp.exp(-(t_lo * t_lo)))
                t_hi = dv - c_hi
                F_buf[r, pl.ds(32, 16)] = _bf16_round(jnp.where(
                    hi_mask, jnp.exp(-(t_hi * t_hi)), zeros16))

    stage(0, sidx_a, didx_a, ea_a, sem_ea)

    def body(k, carry):
        j0 = 2 * k
        j1 = j0 + 1
        stage(j1, sidx_b, didx_b, ea_b, sem_eb)
        stage_wait(j0, sidx_a, didx_a, ea_a, sem_ea)

        @pl.when(k > 0)
        def _():
            pltpu.make_async_copy(F_a, G.at[didx2a.at[0]], sem_sa).wait()

        compute(j0, sidx_a, didx_a, ea_a, F_a, didx2a)
        pltpu.async_copy(F_a, G.at[didx2a.at[0]], sem_sa, add=True)

        @pl.when(k < NCH_S // 2 - 1)
        def _():
            stage(j0 + 2, sidx_a, didx_a, ea_a, sem_ea)

        stage_wait(j1, sidx_b, didx_b, ea_b, sem_eb)

        @pl.when(k > 0)
        def _():
            pltpu.make_async_copy(F_b, G.at[didx2b.at[0]], sem_sb).wait()

        compute(j1, sidx_b, didx_b, ea_b, F_b, didx2b)
        pltpu.async_copy(F_b, G.at[didx2b.at[0]], sem_sb, add=True)
        return carry

    lax.fori_loop(0, NCH_S // 2, body, None)
    pltpu.make_async_copy(F_a, G.at[didx2a.at[0]], sem_sa).wait()
    pltpu.make_async_copy(F_b, G.at[didx2b.at[0]], sem_sb).wait()
    plsc.subcore_barrier()
    pltpu.sync_copy(G.at[pl.ds(base_row, SROWS_PT)], out_hbm.at[c, s])


def _sc_preproc_call(srcS, dstS, posp, eaP, zerosS):
    px, py, pz = posp[0], posp[1], posp[2]
    f = pl.kernel(
        _sc_preproc_body,
        out_type=jax.ShapeDtypeStruct((NC, NS, SROWS_PT, D), _F32),
        mesh=_sc_mesh(),
        scratch_types=[
            pltpu.VMEM((CS,), _I32),
            pltpu.VMEM((CS,), _I32),
            pltpu.VMEM((CS,), _I32),
            pltpu.VMEM((CS,), _I32),
            pltpu.VMEM((1, CS), _I32),
            pltpu.VMEM((1, CS), _I32),
            pltpu.VMEM((NPAD,), _F32),
            pltpu.VMEM((NPAD,), _F32),
            pltpu.VMEM((NPAD,), _F32),
            pltpu.VMEM((CS, 16), _F32),
            pltpu.VMEM((CS, 16), _F32),
            pltpu.VMEM((CS, D), _F32),
            pltpu.VMEM((CS, D), _F32),
            pltpu.VMEM_SHARED((SROWS, D), _F32),
            pltpu.SemaphoreType.DMA,
            pltpu.SemaphoreType.DMA,
            pltpu.SemaphoreType.DMA,
            pltpu.SemaphoreType.DMA,
        ],
        compiler_params=pltpu.CompilerParams(needs_layout_passes=False),
    )
    return f(srcS, dstS, px, py, pz, eaP, zerosS).reshape(NC, SROWS, D)


def _sc_spmm_body(src_hbm, dst_hbm, hw_hbm, zeros_hbm, out_hbm,
                  sidx, didx, didx2, rows_a, rows_b, S, sem_a, sem_b):
    c = lax.axis_index("c")
    s = lax.axis_index("s")
    base_row = s * SROWS_PT
    bound = c * NHALF

    pltpu.sync_copy(zeros_hbm, S.at[pl.ds(base_row, SROWS_PT)])
    pltpu.sync_copy(src_hbm.at[s], sidx)
    pltpu.sync_copy(dst_hbm.at[s], didx)
    plsc.subcore_barrier()

    def remap(j):
        for g in range(CS // 16):
            d = didx[j, pl.ds(g * 16, 16)] - bound
            oob = (d < 0) | (d >= NHALF)
            didx2[0, pl.ds(g * 16, 16)] = jnp.where(oob, NHALF, d)

    pltpu.make_async_copy(hw_hbm.at[sidx.at[0]], rows_a, sem_a).start()

    def body(k, carry):
        j0 = 2 * k
        j1 = j0 + 1
        pltpu.make_async_copy(hw_hbm.at[sidx.at[j1]], rows_b, sem_b).start()
        remap(j0)
        pltpu.make_async_copy(hw_hbm.at[sidx.at[j0]], rows_a, sem_a).wait()
        pltpu.sync_copy(rows_a, S.at[didx2.at[0]], add=True)

        @pl.when(k < NCH_S // 2 - 1)
        def _():
            pltpu.make_async_copy(hw_hbm.at[sidx.at[j0 + 2]], rows_a,
                                  sem_a).start()

        remap(j1)
        pltpu.make_async_copy(hw_hbm.at[sidx.at[j1]], rows_b, sem_b).wait()
        pltpu.sync_copy(rows_b, S.at[didx2.at[0]], add=True)
        return carry

    lax.fori_loop(0, NCH_S // 2, body, None)
    plsc.subcore_barrier()
    pltpu.sync_copy(S.at[pl.ds(base_row, SROWS_PT)], out_hbm.at[c, s])


def _sc_spmm_call(srcS, dstS, hw, zerosS):
    f = pl.kernel(
        _sc_spmm_body,
        out_type=jax.ShapeDtypeStruct((NC, NS, SROWS_PT, D), _F32),
        mesh=_sc_mesh(),
        scratch_types=[
            pltpu.VMEM((NCH_S, CS), _I32),
            pltpu.VMEM((NCH_S, CS), _I32),
            pltpu.VMEM((1, CS), _I32),
            pltpu.VMEM((CS, D), _F32),
            pltpu.VMEM((CS, D), _F32),
            pltpu.VMEM_SHARED((SROWS, D), _F32),
            pltpu.SemaphoreType.DMA,
            pltpu.SemaphoreType.DMA,
        ],
        compiler_params=pltpu.CompilerParams(needs_layout_passes=False),
    )
    return f(srcS, dstS, hw, zerosS).reshape(NC, SROWS, D)


def _tc_prep_body(x_ref, w0_ref, g_ref, wcat_ref, hw_ref, dinv_ref, mall_ref):
    g = g_ref[0]
    deg = g[:, 4:5]
    dinv_ref[...] = jnp.where(deg > 0,
                              1.0 / jnp.sqrt(jnp.maximum(deg, 1e-12)), 0.0)
    mall_ref[...] = jnp.dot(g, wcat_ref[...], preferred_element_type=_F32,
                            precision=lax.Precision.HIGHEST)
    hw_ref[...] = jnp.dot(x_ref[...], w0_ref[...], preferred_element_type=_F32)


def _tc_prep_call(x, w0, g, wcat):
    return pl.pallas_call(
        _tc_prep_body,
        grid=(GRID,),
        in_specs=[
            pl.BlockSpec((BR, D), lambda j: (j, 0)),
            pl.BlockSpec((D, D), lambda j: (0, 0)),
            pl.BlockSpec((1, BR, D), lambda j: (j // (NHALF // BR),
                                                j % (NHALF // BR), 0)),
            pl.BlockSpec((D, NUM_LAYERS * D), lambda j: (0, 0)),
        ],
        out_specs=[
            pl.BlockSpec((BR, D), lambda j: (j, 0)),
            pl.BlockSpec((BR, 1), lambda j: (j, 0)),
            pl.BlockSpec((BR, NUM_LAYERS * D), lambda j: (j, 0)),
        ],
        out_shape=[
            jax.ShapeDtypeStruct((N, D), _F32),
            jax.ShapeDtypeStruct((N, 1), _F32),
            jax.ShapeDtypeStruct((N, NUM_LAYERS * D), _F32),
        ],
    )(x, w0, g, wcat)


def _tc_layer_body_next(s2_ref, mall_ref, dinv_ref, b_ref, bat_ref, wn_ref,
                        hwn_ref, pool_ref):
    agg = dinv_ref[...] * (s2_ref[0] + mall_ref[...])
    h = jnp.maximum(agg + b_ref[...], 0.0)
    onehot_t = (bat_ref[0] == lax.broadcasted_iota(_I32, (N_GRAPHS, BR), 0)
                ).astype(_F32)
    p = jnp.dot(onehot_t, h, preferred_element_type=_F32,
                precision=lax.Precision.HIGHEST)
    j = pl.program_id(0)

    @pl.when(j == 0)
    def _():
        pool_ref[...] = jnp.zeros_like(pool_ref)

    pool_ref[...] += p
    hwn_ref[...] = jnp.dot(h, wn_ref[...], preferred_element_type=_F32)


def _tc_layer_body_last(s2_ref, mall_ref, dinv_ref, b_ref, bat_ref, pool_ref):
    agg = dinv_ref[...] * (s2_ref[0] + mall_ref[...])
    h = jnp.maximum(agg + b_ref[...], 0.0)
    onehot_t = (bat_ref[0] == lax.broadcasted_iota(_I32, (N_GRAPHS, BR), 0)
                ).astype(_F32)
    p = jnp.dot(onehot_t, h, preferred_element_type=_F32,
                precision=lax.Precision.HIGHEST)
    j = pl.program_id(0)

    @pl.when(j == 0)
    def _():
        pool_ref[...] = jnp.zeros_like(pool_ref)

    pool_ref[...] += p


def _tc_layer_call(i, s2, mall, dinv, b, bat3, wn):
    base_in = [
        pl.BlockSpec((1, BR, D), lambda j: (j // (NHALF // BR),
                                            j % (NHALF // BR), 0)),
        pl.BlockSpec((BR, D), lambda j, i=i: (j, i)),
        pl.BlockSpec((BR, 1), lambda j: (j, 0)),
        pl.BlockSpec((1, D), lambda j: (0, 0)),
        pl.BlockSpec((1, 1, BR), lambda j: (j, 0, 0)),
    ]
    if wn is not None:
        return pl.pallas_call(
            _tc_layer_body_next,
            grid=(GRID,),
            in_specs=base_in + [pl.BlockSpec((D, D), lambda j: (0, 0))],
            out_specs=[
                pl.BlockSpec((BR, D), lambda j: (j, 0)),
                pl.BlockSpec((N_GRAPHS, D), lambda j: (0, 0)),
            ],
            out_shape=[
                jax.ShapeDtypeStruct((N, D), _F32),
                jax.ShapeDtypeStruct((N_GRAPHS, D), _F32),
            ],
        )(s2, mall, dinv, b, bat3, wn)
    return pl.pallas_call(
        _tc_layer_body_last,
        grid=(GRID,),
        in_specs=base_in,
        out_specs=[pl.BlockSpec((N_GRAPHS, D), lambda j: (0, 0))],
        out_shape=[jax.ShapeDtypeStruct((N_GRAPHS, D), _F32)],
    )(s2, mall, dinv, b, bat3)


def _tc_head_body(p0_ref, p1_ref, p2_ref, w1_ref, b1_ref, w2_ref, b2_ref,
                  w3_ref, b3_ref, out_ref):
    xa = p0_ref[...] + p1_ref[...] + p2_ref[...]
    z = jnp.maximum(
        jnp.dot(xa, w1_ref[...], preferred_element_type=_F32) + b1_ref[...],
        0.0)
    e = jnp.dot(z, w2_ref[...], preferred_element_type=_F32) + b2_ref[...]
    out_ref[...] = (jnp.dot(jnp.maximum(e, 0.0), w3_ref[...],
                            preferred_element_type=_F32) + b3_ref[...])


def _tc_head_call(p0, p1, p2, params):
    return pl.pallas_call(
        _tc_head_body,
        out_shape=jax.ShapeDtypeStruct((N_GRAPHS, 1), _F32),
    )(p0, p1, p2,
      params['lin1_w'], params['lin1_b'].reshape(1, 64),
      params['lin2_w'], params['lin2_b'].reshape(1, 32),
      params['lin3_w'], params['lin3_b'].reshape(1, 1))


@jax.jit
def kernel(x, edge_index, batch, edge_attr, pos, params):
    src = edge_index[0]
    dst = edge_index[1]
    srcS = src.reshape(NS, NCH_S, CS)
    dstS = dst.reshape(NS, NCH_S, CS)
    posp = jnp.pad(pos.T, ((0, 0), (0, NPAD - N)))
    eaP = jnp.concatenate(
        [edge_attr, jnp.ones((E, 1), _F32), jnp.zeros((E, 11), _F32)],
        axis=1)
    zerosS = jnp.zeros((SROWS_PT, D), _F32)
    bat3 = batch.reshape(GRID, 1, BR)

    wcat = jnp.concatenate([
        jnp.concatenate([
            params['We%d' % i],
            jnp.zeros((12, D), _F32),
            params['Wr%d' % i][:16],
            params['Wr%d' % i][16:],
            jnp.zeros((D - 36, D), _F32),
        ], axis=0) for i in range(NUM_LAYERS)
    ], axis=1)
    wcat = wcat.astype(jnp.bfloat16).astype(_F32)

    g = _sc_preproc_call(src, dst, posp, eaP, zerosS)
    hw, dinv, mall = _tc_prep_call(x, params['W0'], g, wcat)

    pools = []
    for i in range(NUM_LAYERS):
        s2 = _sc_spmm_call(srcS, dstS, hw, zerosS)
        b = params['b%d' % i].reshape(1, D)
        if i < NUM_LAYERS - 1:
            hw, pool_i = _tc_layer_call(i, s2, mall, dinv, b, bat3,
                                        params['W%d' % (i + 1)])
        else:
            (pool_i,) = _tc_layer_call(i, s2, mall, dinv, b, bat3, None)
        pools.append(pool_i)

    return _tc_head_call(pools[0], pools[1], pools[2], params)


# speedup vs baseline: 1.0030x; 1.0030x over previous
import jax
import jax.numpy as jnp
from jax import lax
from jax.experimental import pallas as pl
from jax.experimental.pallas import tpu as pltpu
from jax.experimental.pallas import tpu_sc as plsc

N = 10000
E = 320000
D = 128
N_RBF = 20
N_GRAPHS = 64
NUM_LAYERS = 3

NC = 2
NS = 16
NW = NC * NS
NHALF = N // NC
SROWS = NHALF + 8
SROWS_PT = SROWS // NS
EPT_S = E // NS
CS = 80
NCH_S = EPT_S // CS

BR = 1000
GRID = N // BR

_F32 = jnp.float32
_I32 = jnp.int32


def _bf16_round(v):
    i = plsc.bitcast(v, _I32)
    r = i + 0x8000 + (lax.shift_right_logical(i, jnp.full((16,), 16, _I32))
                      & jnp.full((16,), 1, _I32))
    return plsc.bitcast(r & jnp.full((16,), -65536, _I32), _F32)


def _sc_mesh():
    return plsc.VectorSubcoreMesh(core_axis_name="c", subcore_axis_name="s")


NPAD = 10240


def _sc_preproc_body(src_hbm, dst_hbm, px_hbm, py_hbm, pz_hbm, ea_hbm,
                     zeros_hbm, out_hbm,
                     sidx_a, sidx_b, didx_a, didx_b, didx2a, didx2b,
                     px, py, pz, ea_a, ea_b,
                     F_a, F_b, G, sem_ea, sem_eb, sem_sa, sem_sb):
    c = lax.axis_index("c")
    s = lax.axis_index("s")
    base_row = s * SROWS_PT
    bound = c * NHALF
    ebase = s * EPT_S

    pltpu.sync_copy(zeros_hbm, G.at[pl.ds(base_row, SROWS_PT)])
    pltpu.sync_copy(px_hbm, px)
    pltpu.sync_copy(py_hbm, py)
    pltpu.sync_copy(pz_hbm, pz)
    zeros16 = jnp.zeros((16,), _F32)

    def zero_body(r, carry):
        for blk in range(3, 8):
            F_a[r, pl.ds(16 * blk, 16)] = zeros16
            F_b[r, pl.ds(16 * blk, 16)] = zeros16
        return carry

    lax.fori_loop(0, CS, zero_body, None)
    plsc.subcore_barrier()

    step = 4.0 / (N_RBF - 1)
    iota16 = lax.iota(_I32, 16)
    shift1 = jnp.full((16,), 1, _I32)
    magic = jnp.full((16,), 0x5F3759DF, _I32)
    c_lo = iota16.astype(_F32) * step
    c_hi = (iota16.astype(_F32) + 16.0) * step
    hi_mask = iota16 < (N_RBF - 16)

    def stage(j, sbuf, dbuf, ebuf, sem):
        pltpu.make_async_copy(
            src_hbm.at[pl.ds(ebase + j * CS, CS)], sbuf, sem).start()
        pltpu.make_async_copy(
            dst_hbm.at[pl.ds(ebase + j * CS, CS)], dbuf, sem).start()
        pltpu.make_async_copy(
            ea_hbm.at[pl.ds(ebase + j * CS, CS)], ebuf, sem).start()

    def stage_wait(j, sbuf, dbuf, ebuf, sem):
        pltpu.make_async_copy(
            src_hbm.at[pl.ds(ebase + j * CS, CS)], sbuf, sem).wait()
        pltpu.make_async_copy(
            dst_hbm.at[pl.ds(ebase + j * CS, CS)], dbuf, sem).wait()
        pltpu.make_async_copy(
            ea_hbm.at[pl.ds(ebase + j * CS, CS)], ebuf, sem).wait()

    def compute(j, sidx_buf, didx_buf, ea_buf, F_buf, didx2_buf):
        for g in range(CS // 16):
            sidxv = sidx_buf[pl.ds(g * 16, 16)]
            didxv = didx_buf[pl.ds(g * 16, 16)]
            dx = plsc.load_gather(px, [sidxv]) - plsc.load_gather(px, [didxv])
            dy = plsc.load_gather(py, [sidxv]) - plsc.load_gather(py, [didxv])
            dz = plsc.load_gather(pz, [sidxv]) - plsc.load_gather(pz, [didxv])
            d2v = dx * dx + dy * dy + dz * dz + 1e-12
            bits = plsc.bitcast(d2v, _I32)
            bits = magic - lax.shift_right_logical(bits, shift1)
            y = plsc.bitcast(bits, _F32)
            for _ in range(2):
                y = y * (1.5 - 0.5 * d2v * y * y)
            distv = d2v * y
            d = didxv - bound
            oob = (d < 0) | (d >= NHALF)
            didx2_buf[0, pl.ds(g * 16, 16)] = jnp.where(oob, NHALF, d)
            for e in range(16):
                r = g * 16 + e
                F_buf[r, pl.ds(0, 16)] = _bf16_round(ea_buf[r, pl.ds(0, 16)])
                dv = jnp.broadcast_to(distv[e], (16,))
                t_lo = dv - c_lo
                F_buf[r, pl.ds(16, 16)] = _bf16_round(jnp.exp(-(t_lo * t_lo)))
                t_hi = dv - c_hi
                F_buf[r, pl.ds(32, 16)] = _bf16_round(jnp.where(
                    hi_mask, jnp.exp(-(t_hi * t_hi)), zeros16))

    stage(0, sidx_a, didx_a, ea_a, sem_ea)

    def body(k, carry):
        j0 = 2 * k
        j1 = j0 + 1
        stage(j1, sidx_b, didx_b, ea_b, sem_eb)
        stage_wait(j0, sidx_a, didx_a, ea_a, sem_ea)

        @pl.when(k > 0)
        def _():
            pltpu.make_async_copy(F_a, G.at[didx2a.at[0]], sem_sa).wait()

        compute(j0, sidx_a, didx_a, ea_a, F_a, didx2a)
        pltpu.async_copy(F_a, G.at[didx2a.at[0]], sem_sa, add=True)

        @pl.when(k < NCH_S // 2 - 1)
        def _():
            stage(j0 + 2, sidx_a, didx_a, ea_a, sem_ea)

        stage_wait(j1, sidx_b, didx_b, ea_b, sem_eb)

        @pl.when(k > 0)
        def _():
            pltpu.make_async_copy(F_b, G.at[didx2b.at[0]], sem_sb).wait()

        compute(j1, sidx_b, didx_b, ea_b, F_b, didx2b)
        pltpu.async_copy(F_b, G.at[didx2b.at[0]], sem_sb, add=True)
        return carry

    lax.fori_loop(0, NCH_S // 2, body, None)
    pltpu.make_async_copy(F_a, G.at[didx2a.at[0]], sem_sa).wait()
    pltpu.make_async_copy(F_b, G.at[didx2b.at[0]], sem_sb).wait()
    plsc.subcore_barrier()
    pltpu.sync_copy(G.at[pl.ds(base_row, SROWS_PT)], out_hbm.at[c, s])


def _sc_preproc_call(srcS, dstS, posp, eaP, zerosS):
    px, py, pz = posp[0], posp[1], posp[2]
    f = pl.kernel(
        _sc_preproc_body,
        out_type=jax.ShapeDtypeStruct((NC, NS, SROWS_PT, D), _F32),
        mesh=_sc_mesh(),
        scratch_types=[
            pltpu.VMEM((CS,), _I32),
            pltpu.VMEM((CS,), _I32),
            pltpu.VMEM((CS,), _I32),
            pltpu.VMEM((CS,), _I32),
            pltpu.VMEM((1, CS), _I32),
            pltpu.VMEM((1, CS), _I32),
            pltpu.VMEM((NPAD,), _F32),
            pltpu.VMEM((NPAD,), _F32),
            pltpu.VMEM((NPAD,), _F32),
            pltpu.VMEM((CS, 16), _F32),
            pltpu.VMEM((CS, 16), _F32),
            pltpu.VMEM((CS, D), _F32),
            pltpu.VMEM((CS, D), _F32),
            pltpu.VMEM_SHARED((SROWS, D), _F32),
            pltpu.SemaphoreType.DMA,
            pltpu.SemaphoreType.DMA,
            pltpu.SemaphoreType.DMA,
            pltpu.SemaphoreType.DMA,
        ],
        compiler_params=pltpu.CompilerParams(needs_layout_passes=False),
    )
    return f(srcS, dstS, px, py, pz, eaP, zerosS).reshape(NC, SROWS, D)


def _sc_spmm_body(src_hbm, dst_hbm, hw_hbm, zeros_hbm, out_hbm,
                  sidx, didx, didx2, rows_a, rows_b, S, sem_a, sem_b):
    c = lax.axis_index("c")
    s = lax.axis_index("s")
    base_row = s * SROWS_PT
    bound = c * NHALF

    pltpu.sync_copy(zeros_hbm, S.at[pl.ds(base_row, SROWS_PT)])
    pltpu.sync_copy(src_hbm.at[s], sidx)
    pltpu.sync_copy(dst_hbm.at[s], didx)
    plsc.subcore_barrier()

    def remap(j):
        for g in range(CS // 16):
            d = didx[j, pl.ds(g * 16, 16)] - bound
            oob = (d < 0) | (d >= NHALF)
            didx2[0, pl.ds(g * 16, 16)] = jnp.where(oob, NHALF, d)

    pltpu.make_async_copy(hw_hbm.at[sidx.at[0]], rows_a, sem_a).start()

    def body(k, carry):
        j0 = 2 * k
        j1 = j0 + 1
        pltpu.make_async_copy(hw_hbm.at[sidx.at[j1]], rows_b, sem_b).start()
        remap(j0)
        pltpu.make_async_copy(hw_hbm.at[sidx.at[j0]], rows_a, sem_a).wait()
        pltpu.sync_copy(rows_a, S.at[didx2.at[0]], add=True)

        @pl.when(k < NCH_S // 2 - 1)
        def _():
            pltpu.make_async_copy(hw_hbm.at[sidx.at[j0 + 2]], rows_a,
                                  sem_a).start()

        remap(j1)
        pltpu.make_async_copy(hw_hbm.at[sidx.at[j1]], rows_b, sem_b).wait()
        pltpu.sync_copy(rows_b, S.at[didx2.at[0]], add=True)
        return carry

    lax.fori_loop(0, NCH_S // 2, body, None)
    plsc.subcore_barrier()
    pltpu.sync_copy(S.at[pl.ds(base_row, SROWS_PT)], out_hbm.at[c, s])


def _sc_spmm_call(srcS, dstS, hw, zerosS):
    f = pl.kernel(
        _sc_spmm_body,
        out_type=jax.ShapeDtypeStruct((NC, NS, SROWS_PT, D), _F32),
        mesh=_sc_mesh(),
        scratch_types=[
            pltpu.VMEM((NCH_S, CS), _I32),
            pltpu.VMEM((NCH_S, CS), _I32),
            pltpu.VMEM((1, CS), _I32),
            pltpu.VMEM((CS, D), _F32),
            pltpu.VMEM((CS, D), _F32),
            pltpu.VMEM_SHARED((SROWS, D), _F32),
            pltpu.SemaphoreType.DMA,
            pltpu.SemaphoreType.DMA,
        ],
        compiler_params=pltpu.CompilerParams(needs_layout_passes=False),
    )
    return f(srcS, dstS, hw, zerosS).reshape(NC, SROWS, D)


def _tc_prep_body(x_ref, w0_ref, g_ref, wcat_ref, hw_ref, dinv_ref, mall_ref):
    g = g_ref[0]
    deg = g[:, 4:5]
    dinv_ref[...] = jnp.where(deg > 0,
                              1.0 / jnp.sqrt(jnp.maximum(deg, 1e-12)), 0.0)
    mall_ref[...] = jnp.dot(g, wcat_ref[...], preferred_element_type=_F32,
                            precision=lax.Precision.HIGHEST)
    hw_ref[...] = jnp.dot(x_ref[...], w0_ref[...], preferred_element_type=_F32)


def _tc_prep_call(x, w0, g, wcat):
    return pl.pallas_call(
        _tc_prep_body,
        grid=(GRID,),
        in_specs=[
            pl.BlockSpec((BR, D), lambda j: (j, 0)),
            pl.BlockSpec((D, D), lambda j: (0, 0)),
            pl.BlockSpec((1, BR, D), lambda j: (j // (NHALF // BR),
                                                j % (NHALF // BR), 0)),
            pl.BlockSpec((D, NUM_LAYERS * D), lambda j: (0, 0)),
        ],
        out_specs=[
            pl.BlockSpec((BR, D), lambda j: (j, 0)),
            pl.BlockSpec((BR, 1), lambda j: (j, 0)),
            pl.BlockSpec((BR, NUM_LAYERS * D), lambda j: (j, 0)),
        ],
        out_shape=[
            jax.ShapeDtypeStruct((N, D), _F32),
            jax.ShapeDtypeStruct((N, 1), _F32),
            jax.ShapeDtypeStruct((N, NUM_LAYERS * D), _F32),
        ],
    )(x, w0, g, wcat)


def _tc_layer_body_next(s2_ref, mall_ref, dinv_ref, b_ref, bat_ref, wn_ref,
                        hwn_ref, pool_ref):
    agg = dinv_ref[...] * (s2_ref[0] + mall_ref[...])
    h = jnp.maximum(agg + b_ref[...], 0.0)
    onehot_t = (bat_ref[0] == lax.broadcasted_iota(_I32, (N_GRAPHS, BR), 0)
                ).astype(_F32)
    p = jnp.dot(onehot_t, h, preferred_element_type=_F32,
                precision=lax.Precision.HIGHEST)
    j = pl.program_id(0)

    @pl.when(j == 0)
    def _():
        pool_ref[...] = jnp.zeros_like(pool_ref)

    pool_ref[...] += p
    hwn_ref[...] = jnp.dot(h, wn_ref[...], preferred_element_type=_F32)


def _tc_layer_body_last(s2_ref, mall_ref, dinv_ref, b_ref, bat_ref, pool_ref):
    agg = dinv_ref[...] * (s2_ref[0] + mall_ref[...])
    h = jnp.maximum(agg + b_ref[...], 0.0)
    onehot_t = (bat_ref[0] == lax.broadcasted_iota(_I32, (N_GRAPHS, BR), 0)
                ).astype(_F32)
    p = jnp.dot(onehot_t, h, preferred_element_type=_F32,
                precision=lax.Precision.HIGHEST)
    j = pl.program_id(0)

    @pl.when(j == 0)
    def _():
        pool_ref[...] = jnp.zeros_like(pool_ref)

    pool_ref[...] += p


def _tc_layer_call(i, s2, mall, dinv, b, bat3, wn):
    base_in = [
        pl.BlockSpec((1, BR, D), lambda j: (j // (NHALF // BR),
                                            j % (NHALF // BR), 0)),
        pl.BlockSpec((BR, D), lambda j, i=i: (j, i)),
        pl.BlockSpec((BR, 1), lambda j: (j, 0)),
        pl.BlockSpec((1, D), lambda j: (0, 0)),
        pl.BlockSpec((1, 1, BR), lambda j: (j, 0, 0)),
    ]
    if wn is not None:
        return pl.pallas_call(
            _tc_layer_body_next,
            grid=(GRID,),
            in_specs=base_in + [pl.BlockSpec((D, D), lambda j: (0, 0))],
            out_specs=[
                pl.BlockSpec((BR, D), lambda j: (j, 0)),
                pl.BlockSpec((N_GRAPHS, D), lambda j: (0, 0)),
            ],
            out_shape=[
                jax.ShapeDtypeStruct((N, D), _F32),
                jax.ShapeDtypeStruct((N_GRAPHS, D), _F32),
            ],
        )(s2, mall, dinv, b, bat3, wn)
    return pl.pallas_call(
        _tc_layer_body_last,
        grid=(GRID,),
        in_specs=base_in,
        out_specs=[pl.BlockSpec((N_GRAPHS, D), lambda j: (0, 0))],
        out_shape=[jax.ShapeDtypeStruct((N_GRAPHS, D), _F32)],
    )(s2, mall, dinv, b, bat3)


def _tc_head_body(p0_ref, p1_ref, p2_ref, w1_ref, b1_ref, w2_ref, b2_ref,
                  w3_ref, b3_ref, out_ref):
    xa = p0_ref[...] + p1_ref[...] + p2_ref[...]
    z = jnp.maximum(
        jnp.dot(xa, w1_ref[...], preferred_element_type=_F32) + b1_ref[...],
        0.0)
    e = jnp.dot(z, w2_ref[...], preferred_element_type=_F32) + b2_ref[...]
    out_ref[...] = (jnp.dot(jnp.maximum(e, 0.0), w3_ref[...],
                            preferred_element_type=_F32) + b3_ref[...])


def _tc_head_call(p0, p1, p2, params):
    return pl.pallas_call(
        _tc_head_body,
        out_shape=jax.ShapeDtypeStruct((N_GRAPHS, 1), _F32),
    )(p0, p1, p2,
      params['lin1_w'], params['lin1_b'].reshape(1, 64),
      params['lin2_w'], params['lin2_b'].reshape(1, 32),
      params['lin3_w'], params['lin3_b'].reshape(1, 1))


@jax.jit
def kernel(x, edge_index, batch, edge_attr, pos, params):
    src = edge_index[0]
    dst = edge_index[1]
    srcS = src.reshape(NS, NCH_S, CS)
    dstS = dst.reshape(NS, NCH_S, CS)
    posp = jnp.pad(pos.T, ((0, 0), (0, NPAD - N)))
    eaP = jnp.concatenate(
        [edge_attr, jnp.ones((E, 1), _F32), jnp.zeros((E, 11), _F32)],
        axis=1)
    zerosS = jnp.zeros((SROWS_PT, D), _F32)
    bat3 = batch.reshape(GRID, 1, BR)

    wcat = jnp.concatenate([
        jnp.concatenate([
            params['We%d' % i],
            jnp.zeros((12, D), _F32),
            params['Wr%d' % i][:16],
            params['Wr%d' % i][16:],
            jnp.zeros((D - 36, D), _F32),
        ], axis=0) for i in range(NUM_LAYERS)
    ], axis=1)
    wcat = wcat.astype(jnp.bfloat16).astype(_F32)

    g = _sc_preproc_call(src, dst, posp, eaP, zerosS)
    hw, dinv, mall = _tc_prep_call(x, params['W0'], g, wcat)

    pools = []
    for i in range(NUM_LAYERS):
        s2 = _sc_spmm_call(srcS, dstS, hw, zerosS)
        b = params['b%d' % i].reshape(1, D)
        if i < NUM_LAYERS - 1:
            hw, pool_i = _tc_layer_call(i, s2, mall, dinv, b, bat3,
                                        params['W%d' % (i + 1)])
        else:
            (pool_i,) = _tc_layer_call(i, s2, mall, dinv, b, bat3, None)
        pools.append(pool_i)

    return _tc_head_call(pools[0], pools[1], pools[2], params)
